# baseline (speedup 1.0000x reference)
import jax
import jax.numpy as jnp
from jax.experimental import pallas as pl
from jax.experimental.pallas import tpu as pltpu

_NB = 8
_HC = 64


def _hist_kernel(x_ref, c_ref, w_ref, oned_ref, twod_ref):
    j = pl.program_id(1)
    nj = pl.num_programs(1)

    x = x_ref[0, 0]
    c = c_ref[0, :].reshape(_NB, 1)
    w = w_ref[0, 0]
    W = x.shape[-1]

    @pl.when(j == 0)
    def _init():
        oned_ref[...] = jnp.zeros_like(oned_ref)

    acc = jnp.zeros((_NB, 1), jnp.float32)
    for h in range(_HC):
        xh = x[h:h + 1, :]
        z = w - jnp.abs(xh - c)
        p = jnp.power(jnp.float32(1.01), z)
        xx = jnp.where(p > 1.0, p, jnp.float32(0.0))
        twod_ref[0, :, h * W:(h + 1) * W] = xx
        acc = acc + jnp.sum(xx, axis=1, keepdims=True)
    oned_ref[0, :, :] += acc

    @pl.when(j == nj - 1)
    def _finish():
        oned_ref[...] *= jnp.float32(1.0) / jnp.float32(_HC * W * nj)


def kernel(input_image, centers, width):
    B, C, H, W = input_image.shape
    NB = centers.shape[0]
    HW = H * W
    BC = B * C
    nj = H // _HC
    chunk = _HC * W

    c2 = centers.reshape(1, NB).astype(jnp.float32)
    w2 = jnp.asarray(width, jnp.float32).reshape(1, 1)

    oned, twod = pl.pallas_call(
        _hist_kernel,
        grid=(BC, nj),
        in_specs=[
            pl.BlockSpec((1, 1, _HC, W), lambda i, j: (i // C, i % C, j, 0)),
            pl.BlockSpec((1, NB), lambda i, j: (0, 0)),
            pl.BlockSpec((1, 1), lambda i, j: (0, 0)),
        ],
        out_specs=[
            pl.BlockSpec((1, NB, 1), lambda i, j: (i, 0, 0)),
            pl.BlockSpec((1, NB, chunk), lambda i, j: (i, 0, j)),
        ],
        out_shape=[
            jax.ShapeDtypeStruct((BC, NB, 1), jnp.float32),
            jax.ShapeDtypeStruct((BC, NB, HW), jnp.float32),
        ],
        compiler_params=pltpu.CompilerParams(
            dimension_semantics=("parallel", "arbitrary"),
        ),
    )(input_image, c2, w2)

    one_d = oned.reshape(B, C * NB)
    two_d = twod.reshape(B, C * NB, HW)
    return one_d, two_d


# speedup vs baseline: 1.8075x; 1.8075x over previous
import jax
import jax.numpy as jnp
from jax.experimental import pallas as pl
from jax.experimental.pallas import tpu as pltpu

_NB = 8
_HC = 256


def _hist_kernel(x_ref, c_ref, w_ref, oned_ref, twod_ref):
    j = pl.program_id(1)
    nj = pl.num_programs(1)

    x = x_ref[0, 0]
    c = c_ref[0, :].reshape(_NB, 1)
    w = w_ref[0, 0]
    W = x.shape[-1]

    @pl.when(j == 0)
    def _init():
        oned_ref[...] = jnp.zeros_like(oned_ref)

    acc = jnp.zeros((_NB, 1), jnp.float32)
    for h in range(_HC):
        xh = x[h:h + 1, :]
        z = w - jnp.abs(xh - c)
        p = jnp.power(jnp.float32(1.01), z)
        xx = jnp.where(p > 1.0, p, jnp.float32(0.0))
        twod_ref[0, :, h * W:(h + 1) * W] = xx
        acc = acc + jnp.sum(xx, axis=1, keepdims=True)
    oned_ref[0, :, :] += acc

    @pl.when(j == nj - 1)
    def _finish():
        oned_ref[...] *= jnp.float32(1.0) / jnp.float32(_HC * W * nj)


def kernel(input_image, centers, width):
    B, C, H, W = input_image.shape
    NB = centers.shape[0]
    HW = H * W
    BC = B * C
    nj = H // _HC
    chunk = _HC * W

    c2 = centers.reshape(1, NB).astype(jnp.float32)
    w2 = jnp.asarray(width, jnp.float32).reshape(1, 1)

    oned, twod = pl.pallas_call(
        _hist_kernel,
        grid=(BC, nj),
        in_specs=[
            pl.BlockSpec((1, 1, _HC, W), lambda i, j: (i // C, i % C, j, 0)),
            pl.BlockSpec((1, NB), lambda i, j: (0, 0)),
            pl.BlockSpec((1, 1), lambda i, j: (0, 0)),
        ],
        out_specs=[
            pl.BlockSpec((1, NB, 1), lambda i, j: (i, 0, 0)),
            pl.BlockSpec((1, NB, chunk), lambda i, j: (i, 0, j)),
        ],
        out_shape=[
            jax.ShapeDtypeStruct((BC, NB, 1), jnp.float32),
            jax.ShapeDtypeStruct((BC, NB, HW), jnp.float32),
        ],
        compiler_params=pltpu.CompilerParams(
            dimension_semantics=("parallel", "arbitrary"),
        ),
    )(input_image, c2, w2)

    one_d = oned.reshape(B, C * NB)
    two_d = twod.reshape(B, C * NB, HW)
    return one_d, two_d


# speedup vs baseline: 2.2165x; 1.2263x over previous
import jax
import jax.numpy as jnp
from jax.experimental import pallas as pl
from jax.experimental.pallas import tpu as pltpu

_NB = 8
_HC = 512


def _hist_kernel(x_ref, c_ref, w_ref, oned_ref, twod_ref):
    j = pl.program_id(1)
    nj = pl.num_programs(1)

    x = x_ref[0, 0]
    c = c_ref[0, :].reshape(_NB, 1)
    w = w_ref[0, 0]
    W = x.shape[-1]

    @pl.when(j == 0)
    def _init():
        oned_ref[...] = jnp.zeros_like(oned_ref)

    acc = jnp.zeros((_NB, 1), jnp.float32)
    for h in range(_HC):
        xh = x[h:h + 1, :]
        z = w - jnp.abs(xh - c)
        p = jnp.power(jnp.float32(1.01), z)
        xx = jnp.where(p > 1.0, p, jnp.float32(0.0))
        twod_ref[0, :, h * W:(h + 1) * W] = xx
        acc = acc + jnp.sum(xx, axis=1, keepdims=True)
    oned_ref[0, :, :] += acc

    @pl.when(j == nj - 1)
    def _finish():
        oned_ref[...] *= jnp.float32(1.0) / jnp.float32(_HC * W * nj)


def kernel(input_image, centers, width):
    B, C, H, W = input_image.shape
    NB = centers.shape[0]
    HW = H * W
    BC = B * C
    nj = H // _HC
    chunk = _HC * W

    c2 = centers.reshape(1, NB).astype(jnp.float32)
    w2 = jnp.asarray(width, jnp.float32).reshape(1, 1)

    oned, twod = pl.pallas_call(
        _hist_kernel,
        grid=(BC, nj),
        in_specs=[
            pl.BlockSpec((1, 1, _HC, W), lambda i, j: (i // C, i % C, j, 0)),
            pl.BlockSpec((1, NB), lambda i, j: (0, 0)),
            pl.BlockSpec((1, 1), lambda i, j: (0, 0)),
        ],
        out_specs=[
            pl.BlockSpec((1, NB, 1), lambda i, j: (i, 0, 0)),
            pl.BlockSpec((1, NB, chunk), lambda i, j: (i, 0, j)),
        ],
        out_shape=[
            jax.ShapeDtypeStruct((BC, NB, 1), jnp.float32),
            jax.ShapeDtypeStruct((BC, NB, HW), jnp.float32),
        ],
        compiler_params=pltpu.CompilerParams(
            dimension_semantics=("parallel", "arbitrary"),
        ),
    )(input_image, c2, w2)

    one_d = oned.reshape(B, C * NB)
    two_d = twod.reshape(B, C * NB, HW)
    return one_d, two_d


# speedup vs baseline: 2.2988x; 1.0371x over previous
import jax
import jax.numpy as jnp
from jax.experimental import pallas as pl
from jax.experimental.pallas import tpu as pltpu

_NB = 8
_ROWS = 2


def _hist_kernel(x_ref, c_ref, w_ref, oned_ref, twod_ref):
    c = c_ref[0, :].reshape(_NB, 1)
    w = w_ref[0, 0]
    H, W = x_ref.shape[1], x_ref.shape[2]
    inv = jnp.float32(1.0) / jnp.float32(H * W)

    for k in range(_ROWS):
        acc = jnp.zeros((_NB, 1), jnp.float32)
        for h in range(H):
            xh = x_ref[k, h:h + 1, :]
            z = w - jnp.abs(xh - c)
            p = jnp.power(jnp.float32(1.01), z)
            xx = jnp.where(p > 1.0, p, jnp.float32(0.0))
            twod_ref[k, :, h * W:(h + 1) * W] = xx
            acc = acc + jnp.sum(xx, axis=1, keepdims=True)
        oned_ref[k, :, :] = acc * inv


def kernel(input_image, centers, width):
    B, C, H, W = input_image.shape
    NB = centers.shape[0]
    HW = H * W
    BC = B * C

    x3 = input_image.reshape(BC, H, W)
    c2 = centers.reshape(1, NB).astype(jnp.float32)
    w2 = jnp.asarray(width, jnp.float32).reshape(1, 1)

    oned, twod = pl.pallas_call(
        _hist_kernel,
        grid=(BC // _ROWS,),
        in_specs=[
            pl.BlockSpec((_ROWS, H, W), lambda i: (i, 0, 0)),
            pl.BlockSpec((1, NB), lambda i: (0, 0)),
            pl.BlockSpec((1, 1), lambda i: (0, 0)),
        ],
        out_specs=[
            pl.BlockSpec((_ROWS, NB, 1), lambda i: (i, 0, 0)),
            pl.BlockSpec((_ROWS, NB, HW), lambda i: (i, 0, 0)),
        ],
        out_shape=[
            jax.ShapeDtypeStruct((BC, NB, 1), jnp.float32),
            jax.ShapeDtypeStruct((BC, NB, HW), jnp.float32),
        ],
        compiler_params=pltpu.CompilerParams(
            dimension_semantics=("parallel",),
        ),
    )(x3, c2, w2)

    one_d = oned.reshape(B, C * NB)
    two_d = twod.reshape(B, C * NB, HW)
    return one_d, two_d


# speedup vs baseline: 2.3155x; 1.0073x over previous
import jax
import jax.numpy as jnp
from jax.experimental import pallas as pl
from jax.experimental.pallas import tpu as pltpu

_NB = 8
_ROWS = 3


def _hist_kernel(x_ref, c_ref, w_ref, oned_ref, twod_ref):
    c = c_ref[0, :].reshape(_NB, 1)
    w = w_ref[0, 0]
    H, W = x_ref.shape[1], x_ref.shape[2]
    inv = jnp.float32(1.0) / jnp.float32(H * W)

    for k in range(_ROWS):
        acc = jnp.zeros((_NB, 1), jnp.float32)
        for h in range(H):
            xh = x_ref[k, h:h + 1, :]
            z = w - jnp.abs(xh - c)
            p = jnp.power(jnp.float32(1.01), z)
            xx = jnp.where(p > 1.0, p, jnp.float32(0.0))
            twod_ref[k, :, h * W:(h + 1) * W] = xx
            acc = acc + jnp.sum(xx, axis=1, keepdims=True)
        oned_ref[k, :, :] = acc * inv


def kernel(input_image, centers, width):
    B, C, H, W = input_image.shape
    NB = centers.shape[0]
    HW = H * W
    BC = B * C

    x3 = input_image.reshape(BC, H, W)
    c2 = centers.reshape(1, NB).astype(jnp.float32)
    w2 = jnp.asarray(width, jnp.float32).reshape(1, 1)

    oned, twod = pl.pallas_call(
        _hist_kernel,
        grid=(BC // _ROWS,),
        in_specs=[
            pl.BlockSpec((_ROWS, H, W), lambda i: (i, 0, 0)),
            pl.BlockSpec((1, NB), lambda i: (0, 0)),
            pl.BlockSpec((1, 1), lambda i: (0, 0)),
        ],
        out_specs=[
            pl.BlockSpec((_ROWS, NB, 1), lambda i: (i, 0, 0)),
            pl.BlockSpec((_ROWS, NB, HW), lambda i: (i, 0, 0)),
        ],
        out_shape=[
            jax.ShapeDtypeStruct((BC, NB, 1), jnp.float32),
            jax.ShapeDtypeStruct((BC, NB, HW), jnp.float32),
        ],
        compiler_params=pltpu.CompilerParams(
            dimension_semantics=("parallel",),
        ),
    )(x3, c2, w2)

    one_d = oned.reshape(B, C * NB)
    two_d = twod.reshape(B, C * NB, HW)
    return one_d, two_d


# speedup vs baseline: 2.4571x; 1.0611x over previous
import jax
import jax.numpy as jnp
from jax.experimental import pallas as pl
from jax.experimental.pallas import tpu as pltpu

_NB = 8
_ROWS = 3
_LOG2_BASE = 0.014355292977070041


def _hist_kernel(x_ref, c_ref, w_ref, oned_ref, twod_ref):
    c = c_ref[0, :].reshape(_NB, 1)
    w = w_ref[0, 0]
    H, W = x_ref.shape[1], x_ref.shape[2]
    inv = jnp.float32(1.0) / jnp.float32(H * W)

    for k in range(_ROWS):
        acc = jnp.zeros((_NB, 1), jnp.float32)
        for h in range(H):
            xh = x_ref[k, h:h + 1, :]
            z = w - jnp.abs(xh - c)
            p = jnp.exp2(z * jnp.float32(_LOG2_BASE))
            xx = jnp.where(p > 1.0, p, jnp.float32(0.0))
            twod_ref[k, :, h * W:(h + 1) * W] = xx
            acc = acc + jnp.sum(xx, axis=1, keepdims=True)
        oned_ref[k, :, :] = acc * inv


def kernel(input_image, centers, width):
    B, C, H, W = input_image.shape
    NB = centers.shape[0]
    HW = H * W
    BC = B * C

    x3 = input_image.reshape(BC, H, W)
    c2 = centers.reshape(1, NB).astype(jnp.float32)
    w2 = jnp.asarray(width, jnp.float32).reshape(1, 1)

    oned, twod = pl.pallas_call(
        _hist_kernel,
        grid=(BC // _ROWS,),
        in_specs=[
            pl.BlockSpec((_ROWS, H, W), lambda i: (i, 0, 0)),
            pl.BlockSpec((1, NB), lambda i: (0, 0)),
            pl.BlockSpec((1, 1), lambda i: (0, 0)),
        ],
        out_specs=[
            pl.BlockSpec((_ROWS, NB, 1), lambda i: (i, 0, 0)),
            pl.BlockSpec((_ROWS, NB, HW), lambda i: (i, 0, 0)),
        ],
        out_shape=[
            jax.ShapeDtypeStruct((BC, NB, 1), jnp.float32),
            jax.ShapeDtypeStruct((BC, NB, HW), jnp.float32),
        ],
        compiler_params=pltpu.CompilerParams(
            dimension_semantics=("parallel",),
        ),
    )(x3, c2, w2)

    one_d = oned.reshape(B, C * NB)
    two_d = twod.reshape(B, C * NB, HW)
    return one_d, two_d


# speedup vs baseline: 2.4597x; 1.0011x over previous
import jax
import jax.numpy as jnp
from jax.experimental import pallas as pl
from jax.experimental.pallas import tpu as pltpu

_NB = 8
_ROWS = 3
_LOG2_BASE = 0.014355292977070041


def _hist_kernel(x_ref, c_ref, w_ref, oned_ref, twod_ref):
    c = c_ref[0, :].reshape(_NB, 1)
    w = w_ref[0, 0]
    H, W = x_ref.shape[1], x_ref.shape[2]
    inv = jnp.float32(1.0) / jnp.float32(H * W)

    for k in range(_ROWS):
        accv = jnp.zeros((_NB, x_ref.shape[2]), jnp.float32)
        for h in range(H):
            xh = x_ref[k, h:h + 1, :]
            z = w - jnp.abs(xh - c)
            p = jnp.exp2(z * jnp.float32(_LOG2_BASE))
            xx = jnp.where(p > 1.0, p, jnp.float32(0.0))
            twod_ref[k, :, h * W:(h + 1) * W] = xx
            accv = accv + xx
        oned_ref[k, :, :] = jnp.sum(accv, axis=1, keepdims=True) * inv


def kernel(input_image, centers, width):
    B, C, H, W = input_image.shape
    NB = centers.shape[0]
    HW = H * W
    BC = B * C

    x3 = input_image.reshape(BC, H, W)
    c2 = centers.reshape(1, NB).astype(jnp.float32)
    w2 = jnp.asarray(width, jnp.float32).reshape(1, 1)

    oned, twod = pl.pallas_call(
        _hist_kernel,
        grid=(BC // _ROWS,),
        in_specs=[
            pl.BlockSpec((_ROWS, H, W), lambda i: (i, 0, 0)),
            pl.BlockSpec((1, NB), lambda i: (0, 0)),
            pl.BlockSpec((1, 1), lambda i: (0, 0)),
        ],
        out_specs=[
            pl.BlockSpec((_ROWS, NB, 1), lambda i: (i, 0, 0)),
            pl.BlockSpec((_ROWS, NB, HW), lambda i: (i, 0, 0)),
        ],
        out_shape=[
            jax.ShapeDtypeStruct((BC, NB, 1), jnp.float32),
            jax.ShapeDtypeStruct((BC, NB, HW), jnp.float32),
        ],
        compiler_params=pltpu.CompilerParams(
            dimension_semantics=("parallel",),
        ),
    )(x3, c2, w2)

    one_d = oned.reshape(B, C * NB)
    two_d = twod.reshape(B, C * NB, HW)
    return one_d, two_d
